# baseline (speedup 1.0000x reference)
import jax
import jax.numpy as jnp
from jax.experimental import pallas as pl
from jax.experimental.pallas import tpu as pltpu

_TAU = 0.07


def _stage1(mask_ref, fq_ref, fk_ref, sq_ref, sk_ref, cnt_ref):
    k = pl.program_id(1)
    m = mask_ref[0]
    fq = fq_ref[0]
    fk = fk_ref[0]
    dn = (((1,), (1,)), ((), ()))
    sq = jax.lax.dot_general(m, fq, dn, preferred_element_type=jnp.float32)
    sk = jax.lax.dot_general(m, fk, dn, preferred_element_type=jnp.float32)
    cnt = jnp.sum(m, axis=1, keepdims=True)

    @pl.when(k == 0)
    def _init():
        sq_ref[0] = sq
        sk_ref[0] = sk
        cnt_ref[0] = cnt

    @pl.when(k != 0)
    def _acc():
        sq_ref[0] += sq
        sk_ref[0] += sk
        cnt_ref[0] += cnt


def _stage2(sq_ref, sk_ref, cnt_ref, out_ref):
    n = sq_ref.shape[0]
    cnt = jnp.maximum(cnt_ref[...], 1.0)
    mq = sq_ref[...] / cnt
    mk = sk_ref[...] / cnt
    pad = (mk[:, 0:1] != 0).astype(jnp.float32)

    nq = mq / jnp.maximum(
        jnp.sqrt(jnp.sum(mq * mq, axis=-1, keepdims=True)), 1e-12)
    nk = mk / jnp.maximum(
        jnp.sqrt(jnp.sum(mk * mk, axis=-1, keepdims=True)), 1e-12)

    dn = (((1,), (1,)), ((), ()))
    rows = jax.lax.dot_general(nk, nq, dn,
                               preferred_element_type=jnp.float32) / _TAU
    mx = jnp.max(rows, axis=-1, keepdims=True)
    lse = jnp.log(jnp.sum(jnp.exp(rows - mx), axis=-1, keepdims=True)) + mx
    ii = jax.lax.broadcasted_iota(jnp.int32, (n, n), 0)
    jj = jax.lax.broadcasted_iota(jnp.int32, (n, n), 1)
    diag = jnp.sum(jnp.where(ii == jj, rows, 0.0), axis=-1, keepdims=True)
    ce = lse - diag
    num = jnp.sum(ce * pad)
    den = jnp.maximum(jnp.sum(pad), 1.0)
    out_ref[...] = jnp.reshape(num / den, (1, 1))


def kernel(features_q, features_k, pos_region_ranges):
    b, c, h, w = features_q.shape
    mnum = pos_region_ranges.shape[1]
    hw = h * w
    hwc = 2048
    kk = hw // hwc

    maskf = pos_region_ranges.reshape(b, mnum, hw).astype(jnp.float32)
    fq = features_q.reshape(b, c, hw)
    fk = features_k.reshape(b, c, hw)

    sq, sk, cnt = pl.pallas_call(
        _stage1,
        grid=(b, kk),
        in_specs=[
            pl.BlockSpec((1, mnum, hwc), lambda i, j: (i, 0, j)),
            pl.BlockSpec((1, c, hwc), lambda i, j: (i, 0, j)),
            pl.BlockSpec((1, c, hwc), lambda i, j: (i, 0, j)),
        ],
        out_specs=[
            pl.BlockSpec((1, mnum, c), lambda i, j: (i, 0, 0)),
            pl.BlockSpec((1, mnum, c), lambda i, j: (i, 0, 0)),
            pl.BlockSpec((1, mnum, 1), lambda i, j: (i, 0, 0)),
        ],
        out_shape=[
            jax.ShapeDtypeStruct((b, mnum, c), jnp.float32),
            jax.ShapeDtypeStruct((b, mnum, c), jnp.float32),
            jax.ShapeDtypeStruct((b, mnum, 1), jnp.float32),
        ],
        compiler_params=pltpu.CompilerParams(
            dimension_semantics=("parallel", "arbitrary")),
    )(maskf, fq, fk)

    n = b * mnum
    loss = pl.pallas_call(
        _stage2,
        out_shape=jax.ShapeDtypeStruct((1, 1), jnp.float32),
    )(sq.reshape(n, c), sk.reshape(n, c), cnt.reshape(n, 1))
    return loss[0, 0]


# speedup vs baseline: 1.1210x; 1.1210x over previous
import jax
import jax.numpy as jnp
from jax.experimental import pallas as pl
from jax.experimental.pallas import tpu as pltpu

_TAU = 0.07


def _stage1(mask_ref, fq_ref, fk_ref, sq_ref, sk_ref, cnt_ref):
    k = pl.program_id(1)
    m = mask_ref[0]
    fq = fq_ref[0]
    fk = fk_ref[0]
    dn = (((1,), (1,)), ((), ()))
    sq = jax.lax.dot_general(m, fq, dn, preferred_element_type=jnp.float32)
    sk = jax.lax.dot_general(m, fk, dn, preferred_element_type=jnp.float32)
    cnt = jnp.sum(m, axis=1, keepdims=True)

    @pl.when(k == 0)
    def _init():
        sq_ref[0] = sq
        sk_ref[0] = sk
        cnt_ref[0] = cnt

    @pl.when(k != 0)
    def _acc():
        sq_ref[0] += sq
        sk_ref[0] += sk
        cnt_ref[0] += cnt


def _stage2(sq_ref, sk_ref, cnt_ref, out_ref):
    n = sq_ref.shape[0]
    cnt = jnp.maximum(cnt_ref[...], 1.0)
    mq = sq_ref[...] / cnt
    mk = sk_ref[...] / cnt
    pad = (mk[:, 0:1] != 0).astype(jnp.float32)

    nq = mq / jnp.maximum(
        jnp.sqrt(jnp.sum(mq * mq, axis=-1, keepdims=True)), 1e-12)
    nk = mk / jnp.maximum(
        jnp.sqrt(jnp.sum(mk * mk, axis=-1, keepdims=True)), 1e-12)

    dn = (((1,), (1,)), ((), ()))
    rows = jax.lax.dot_general(nk, nq, dn,
                               preferred_element_type=jnp.float32) / _TAU
    mx = jnp.max(rows, axis=-1, keepdims=True)
    lse = jnp.log(jnp.sum(jnp.exp(rows - mx), axis=-1, keepdims=True)) + mx
    ii = jax.lax.broadcasted_iota(jnp.int32, (n, n), 0)
    jj = jax.lax.broadcasted_iota(jnp.int32, (n, n), 1)
    diag = jnp.sum(jnp.where(ii == jj, rows, 0.0), axis=-1, keepdims=True)
    ce = lse - diag
    num = jnp.sum(ce * pad)
    den = jnp.maximum(jnp.sum(pad), 1.0)
    out_ref[...] = jnp.reshape(num / den, (1, 1))


def kernel(features_q, features_k, pos_region_ranges):
    b, c, h, w = features_q.shape
    mnum = pos_region_ranges.shape[1]
    hw = h * w
    hwc = 8192
    kk = hw // hwc

    maskf = pos_region_ranges.reshape(b, mnum, hw).astype(jnp.float32)
    fq = features_q.reshape(b, c, hw)
    fk = features_k.reshape(b, c, hw)

    sq, sk, cnt = pl.pallas_call(
        _stage1,
        grid=(b, kk),
        in_specs=[
            pl.BlockSpec((1, mnum, hwc), lambda i, j: (i, 0, j)),
            pl.BlockSpec((1, c, hwc), lambda i, j: (i, 0, j)),
            pl.BlockSpec((1, c, hwc), lambda i, j: (i, 0, j)),
        ],
        out_specs=[
            pl.BlockSpec((1, mnum, c), lambda i, j: (i, 0, 0)),
            pl.BlockSpec((1, mnum, c), lambda i, j: (i, 0, 0)),
            pl.BlockSpec((1, mnum, 1), lambda i, j: (i, 0, 0)),
        ],
        out_shape=[
            jax.ShapeDtypeStruct((b, mnum, c), jnp.float32),
            jax.ShapeDtypeStruct((b, mnum, c), jnp.float32),
            jax.ShapeDtypeStruct((b, mnum, 1), jnp.float32),
        ],
        compiler_params=pltpu.CompilerParams(
            dimension_semantics=("parallel", "arbitrary")),
    )(maskf, fq, fk)

    n = b * mnum
    loss = pl.pallas_call(
        _stage2,
        out_shape=jax.ShapeDtypeStruct((1, 1), jnp.float32),
    )(sq.reshape(n, c), sk.reshape(n, c), cnt.reshape(n, 1))
    return loss[0, 0]
